# baseline (speedup 1.0000x reference)
import jax
import jax.numpy as jnp
from jax import lax
from jax.experimental import pallas as pl
from jax.experimental.pallas import tpu as pltpu
from jax.experimental.pallas import tpu_sc as plsc

_S = 64.0
_M2 = 0.4
_B = 4096
_V = 100000

_NC, _NS, _L = 2, 16, 16
_NW = _NC * _NS
_PW = _B // _NW


def _sc_body(logits_hbm, labels_hbm, upd_hbm, lab_v, idx_v, val_v, sem):
    wid = lax.axis_index("s") * _NC + lax.axis_index("c")
    base = wid * _PW
    pltpu.sync_copy(labels_hbm.at[pl.ds(base, _PW)], lab_v)
    for j in range(_PW // _L):
        lab = lab_v[pl.ds(j * _L, _L)]
        rows = base + j * _L + lax.iota(jnp.int32, _L)
        safe = jnp.maximum(lab, 0)
        idx_v[pl.ds(j * _L, _L)] = rows * _V + safe
    pltpu.async_copy(logits_hbm.at[idx_v], val_v, sem).wait()
    for j in range(_PW // _L):
        x = val_v[pl.ds(j * _L, _L)]
        a = jnp.maximum(1.0 - x * x, 0.0)
        bits = lax.bitcast_convert_type(a, jnp.int32)
        bits = 0x5F3759DF - lax.shift_right_logical(bits, 1)
        r = lax.bitcast_convert_type(bits, jnp.float32)
        r = r * (1.5 - 0.5 * a * r * r)
        r = r * (1.5 - 0.5 * a * r * r)
        r = r * (1.5 - 0.5 * a * r * r)
        s = a * r
        val_v[pl.ds(j * _L, _L)] = (x / (1.0 + s) - _M2) * _S
    pltpu.sync_copy(val_v, upd_hbm.at[pl.ds(base, _PW)])


_sc_margin_cache = []


def _sc_margin():
    if not _sc_margin_cache:
        _sc_margin_cache.append(pl.kernel(
            _sc_body,
            out_type=jax.ShapeDtypeStruct((_B,), jnp.float32),
            mesh=plsc.VectorSubcoreMesh(core_axis_name="c", subcore_axis_name="s"),
            scratch_types=[
                pltpu.VMEM((_PW,), jnp.int32),
                pltpu.VMEM((_PW,), jnp.int32),
                pltpu.VMEM((_PW,), jnp.float32),
                pltpu.SemaphoreType.DMA,
            ],
        ))
    return _sc_margin_cache[0]

_BR, _BC = 256, 2048
_GR, _GC = _B // _BR, -(-_V // _BC)


def _tc_body(x_ref, lab_ref, upd_ref, o_ref):
    j = pl.program_id(1)
    lab = lab_ref[0]
    upd = upd_ref[0]
    cols = j * _BC + lax.broadcasted_iota(jnp.int32, (_BR, _BC), 1)
    o_ref[...] = jnp.where(cols == lab, upd, x_ref[...] * _S)


_tc_scale = pl.pallas_call(
    _tc_body,
    grid=(_GR, _GC),
    in_specs=[
        pl.BlockSpec((_BR, _BC), lambda i, j: (i, j)),
        pl.BlockSpec((1, _BR, 1), lambda i, j: (i, 0, 0)),
        pl.BlockSpec((1, _BR, 1), lambda i, j: (i, 0, 0)),
    ],
    out_specs=pl.BlockSpec((_BR, _BC), lambda i, j: (i, j)),
    out_shape=jax.ShapeDtypeStruct((_B, _V), jnp.float32),
    compiler_params=pltpu.CompilerParams(
        dimension_semantics=("parallel", "parallel"),
    ),
)


def kernel(logits, labels):
    upd = _sc_margin()(logits.reshape(-1), labels)
    lab3 = labels.reshape(_GR, _BR, 1)
    upd3 = upd.reshape(_GR, _BR, 1)
    return _tc_scale(logits, lab3, upd3)
